# baseline (speedup 1.0000x reference)
import functools

import jax
import jax.numpy as jnp
from jax import lax
from jax.experimental import pallas as pl
from jax.experimental.pallas import tpu as pltpu
from jax.experimental.pallas import tpu_sc as plsc

BATCH = 16384
EMB = 32
NC = 2
NS = 16
NW = NC * NS
BPW = BATCH // NW
CHUNK = 128
NCHUNK = BPW // CHUNK
GROUPS = BPW // 16


def _mf_body(u2, v2, ue, ve, bu, bv, out,
             idx_u, idx_v, rows_u, rows_v, bias_u, bias_v, obuf, sem):
    wid = lax.axis_index("s") * NC + lax.axis_index("c")
    base = wid * BPW

    pltpu.sync_copy(u2.at[pl.ds(wid * NCHUNK, NCHUNK)], idx_u)
    pltpu.sync_copy(v2.at[pl.ds(wid * NCHUNK, NCHUNK)], idx_v)

    descs = []
    for k in range(NCHUNK):
        sl = pl.ds(k * CHUNK, CHUNK)
        descs.append(pltpu.async_copy(ue.at[idx_u.at[k]], rows_u.at[sl], sem))
        descs.append(pltpu.async_copy(ve.at[idx_v.at[k]], rows_v.at[sl], sem))
        descs.append(pltpu.async_copy(bu.at[idx_u.at[k]], bias_u.at[sl], sem))
        descs.append(pltpu.async_copy(bv.at[idx_v.at[k]], bias_v.at[sl], sem))
    for d in descs:
        d.wait()

    lanes = lax.iota(jnp.int32, 16)

    def group(g, carry):
        r0 = pl.multiple_of(g * 16, 16)
        rows = r0 + lanes
        acc = bias_u[pl.ds(r0, 16)] + bias_v[pl.ds(r0, 16)]
        for j in range(EMB):
            col = jnp.full((16,), j, jnp.int32)
            uu = plsc.load_gather(rows_u, [rows, col])
            vv = plsc.load_gather(rows_v, [rows, col])
            acc = acc + uu * vv
        obuf[pl.ds(r0, 16)] = acc
        return carry

    lax.fori_loop(0, GROUPS, group, 0)
    pltpu.sync_copy(obuf, out.at[pl.ds(base, BPW)])


def kernel(u, v, user_emb, movie_emb, user_bias, movie_bias):
    mesh = plsc.VectorSubcoreMesh(core_axis_name="c", subcore_axis_name="s")
    run = functools.partial(
        pl.kernel,
        mesh=mesh,
        compiler_params=pltpu.CompilerParams(
            needs_layout_passes=False, use_tc_tiling_on_sc=False),
        out_type=jax.ShapeDtypeStruct((BATCH,), jnp.float32),
        scratch_types=[
            pltpu.VMEM((NCHUNK, CHUNK), jnp.int32),
            pltpu.VMEM((NCHUNK, CHUNK), jnp.int32),
            pltpu.VMEM((BPW, EMB), jnp.float32),
            pltpu.VMEM((BPW, EMB), jnp.float32),
            pltpu.VMEM((BPW,), jnp.float32),
            pltpu.VMEM((BPW,), jnp.float32),
            pltpu.VMEM((BPW,), jnp.float32),
            pltpu.SemaphoreType.DMA,
        ],
    )(_mf_body)
    u2 = u.reshape(NW * NCHUNK, CHUNK)
    v2 = v.reshape(NW * NCHUNK, CHUNK)
    return run(u2, v2, user_emb, movie_emb,
               user_bias.reshape(-1), movie_bias.reshape(-1))


# speedup vs baseline: 2.7949x; 2.7949x over previous
import functools

import jax
import jax.numpy as jnp
from jax import lax
from jax.experimental import pallas as pl
from jax.experimental.pallas import tpu as pltpu
from jax.experimental.pallas import tpu_sc as plsc

BATCH = 16384
EMB = 32
NC = 2
NS = 16
NW = NC * NS
BPW = BATCH // NW
CHUNK = 16
NCHUNK = BPW // CHUNK


def _mf_body(u1, v1, ue_t, ve_t, bu, bv, out,
             idx_u, idx_v, buf, bias_u, bias_v, obuf, sem, bsem):
    wid = lax.axis_index("s") * NC + lax.axis_index("c")
    base = wid * BPW

    pltpu.sync_copy(u1.at[pl.ds(base, BPW)], idx_u)
    pltpu.sync_copy(v1.at[pl.ds(base, BPW)], idx_v)

    bdescs = []
    for k in range(4):
        sl = pl.ds(k * 128, 128)
        bdescs.append(pltpu.async_copy(bu.at[idx_u.at[sl]], bias_u.at[sl], bsem))
        bdescs.append(pltpu.async_copy(bv.at[idx_v.at[sl]], bias_v.at[sl], bsem))

    lanes = lax.iota(jnp.int32, 16)
    mask127 = jnp.full((16,), 127, jnp.int32)

    def chunk_body(c, carry):
        r0 = pl.multiple_of(c * CHUNK, CHUNK)

        ivec_u = idx_u[pl.ds(r0, 16)]
        ivec_v = idx_v[pl.ds(r0, 16)]
        cvec_u = (ivec_u >> 7) << 7
        cvec_v = (ivec_v >> 7) << 7

        descs = []
        for r in range(CHUNK):
            cu = pl.multiple_of(cvec_u[r], 128)
            descs.append(pltpu.async_copy(
                ue_t.at[:, pl.ds(cu, 128)], buf.at[r], sem))
        for d in descs:
            d.wait()
        colu = jnp.bitwise_and(ivec_u, mask127)
        uvals = []
        for e in range(EMB):
            esplat = jnp.full((16,), e, jnp.int32)
            uvals.append(plsc.load_gather(buf, [lanes, esplat, colu]))

        descs = []
        for r in range(CHUNK):
            cv = pl.multiple_of(cvec_v[r], 128)
            descs.append(pltpu.async_copy(
                ve_t.at[:, pl.ds(cv, 128)], buf.at[r], sem))
        for d in descs:
            d.wait()
        colv = jnp.bitwise_and(ivec_v, mask127)
        acc = jnp.zeros((16,), jnp.float32)
        for e in range(EMB):
            esplat = jnp.full((16,), e, jnp.int32)
            acc = acc + uvals[e] * plsc.load_gather(buf, [lanes, esplat, colv])

        obuf[pl.ds(r0, 16)] = acc
        return carry

    lax.fori_loop(0, NCHUNK, chunk_body, 0)

    for d in bdescs:
        d.wait()

    def addb(g, carry):
        r0 = pl.multiple_of(g * 16, 16)
        obuf[pl.ds(r0, 16)] = (obuf[pl.ds(r0, 16)]
                               + bias_u[pl.ds(r0, 16)] + bias_v[pl.ds(r0, 16)])
        return carry

    lax.fori_loop(0, BPW // 16, addb, 0)
    pltpu.sync_copy(obuf, out.at[pl.ds(base, BPW)])


def kernel(u, v, user_emb, movie_emb, user_bias, movie_bias):
    mesh = plsc.VectorSubcoreMesh(core_axis_name="c", subcore_axis_name="s")
    run = functools.partial(
        pl.kernel,
        mesh=mesh,
        compiler_params=pltpu.CompilerParams(
            needs_layout_passes=False, use_tc_tiling_on_sc=True),
        out_type=jax.ShapeDtypeStruct((BATCH,), jnp.float32),
        scratch_types=[
            pltpu.VMEM((BPW,), jnp.int32),
            pltpu.VMEM((BPW,), jnp.int32),
            pltpu.VMEM((CHUNK, EMB, 128), jnp.float32),
            pltpu.VMEM((BPW,), jnp.float32),
            pltpu.VMEM((BPW,), jnp.float32),
            pltpu.VMEM((BPW,), jnp.float32),
            pltpu.SemaphoreType.DMA,
            pltpu.SemaphoreType.DMA,
        ],
    )(_mf_body)
    return run(u, v, user_emb.T, movie_emb.T,
               user_bias.reshape(-1), movie_bias.reshape(-1))
